# baseline (speedup 1.0000x reference)
import functools

import jax
import jax.numpy as jnp
from jax import lax
from jax.experimental import pallas as pl
from jax.experimental.pallas import tpu as pltpu
from jax.experimental.pallas import tpu_sc as plsc

N = 10000
E = 320000
D = 128
R = 16
B = 8
NR = N * R
NRP = NR + 16
EPT = 20480
EP = EPT * 16
NBP = EPT // 128
CH = 1024


def _sc_layer_kernel(xvp, esrc, edst, et, z2, o2, s_out, c_out,
                     eb1, eb2, seg2d, idx2d, gbuf,
                     sacc, sm0, sm1, sm2, sm3):
  c = lax.axis_index("c")
  sid = lax.axis_index("s")
  base = sid * EPT
  gbase = c * 8
  sems = (sm0, sm1, sm2, sm3)

  for ch in range(EPT // CH):
    off = base + ch * CH
    pltpu.sync_copy(edst.at[pl.ds(off, CH)], eb1)
    pltpu.sync_copy(et.at[pl.ds(off, CH)], eb2)

    def segb(i, _, ch=ch):
      row = ch * 8 + lax.shift_right_logical(i, 3)
      col = (i & 7) * 16
      seg2d[row, pl.ds(col, 16)] = (
          eb1[pl.ds(i * 16, 16)] * R + eb2[pl.ds(i * 16, 16)])
      return _
    lax.fori_loop(0, CH // 16, segb, 0)

    pltpu.sync_copy(esrc.at[pl.ds(off, CH)], eb1)

    def idxb(i, _, ch=ch):
      row = ch * 8 + lax.shift_right_logical(i, 3)
      col = (i & 7) * 16
      idx2d[row, pl.ds(col, 16)] = eb1[pl.ds(i * 16, 16)] * 16 + gbase
      return _
    lax.fori_loop(0, CH // 16, idxb, 0)

  def zero_own():
    pltpu.sync_copy(z2, sacc.at[pl.ds(sid * 10000, 10000)])

  @pl.when(c == 0)
  def _():
    zero_own()
    pltpu.sync_copy(o2, gbuf.at[0])
    plsc.subcore_barrier()

    def cfire(b, u):
      pltpu.async_copy(gbuf.at[0], sacc.at[seg2d.at[b]], sems[u], add=True)

    def cwait(b, u):
      pltpu.make_async_copy(gbuf.at[0], sacc.at[seg2d.at[b]],
                            sems[u]).wait()

    for k in range(4):
      cfire(k, k)

    def cnt(q, _):
      for k in range(4):
        b = 4 * q + k
        cwait(b - 4, k)
        cfire(b, k)
      return _
    lax.fori_loop(1, NBP // 4, cnt, 0)
    for k in range(4):
      cwait(NBP - 4 + k, k)
    plsc.subcore_barrier()
    pltpu.sync_copy(sacc.at[pl.ds(sid * 10000, 10000)],
                    c_out.at[pl.ds(sid * 10000, 10000)])

  for p in range(8):
    table = xvp.at[pl.ds(p, NR)]
    zero_own()
    plsc.subcore_barrier()

    def fire_g(b, u):
      pltpu.async_copy(table.at[idx2d.at[b]], gbuf.at[u], sems[u])

    def wait_g(b, u):
      pltpu.make_async_copy(table.at[idx2d.at[b]], gbuf.at[u],
                            sems[u]).wait()

    def fire_s(b, u):
      pltpu.async_copy(gbuf.at[u], sacc.at[seg2d.at[b]], sems[u], add=True)

    def wait_s(b, u):
      pltpu.make_async_copy(gbuf.at[u], sacc.at[seg2d.at[b]],
                            sems[u]).wait()

    fire_g(0, 0)
    fire_g(1, 1)
    wait_g(0, 0)
    fire_s(0, 0)
    fire_g(2, 2)
    wait_g(1, 1)
    fire_s(1, 1)
    fire_g(3, 3)

    def ring(tt, _):
      for k in range(4):
        t = 4 * tt + 2 + k
        u = (2 + k) % 4
        wait_g(t, u)
        fire_s(t, u)
        wait_s(t - 2, k % 4)
        fire_g(t + 2, k % 4)
      return _
    lax.fori_loop(0, (NBP - 4) // 4, ring, 0)

    wait_g(NBP - 2, 2)
    fire_s(NBP - 2, 2)
    wait_g(NBP - 1, 3)
    fire_s(NBP - 1, 3)
    wait_s(NBP - 4, 0)
    wait_s(NBP - 3, 1)
    wait_s(NBP - 2, 2)
    wait_s(NBP - 1, 3)

    plsc.subcore_barrier()
    pltpu.sync_copy(sacc.at[pl.ds(sid * 10000, 10000)],
                    s_out.at[c * 8 + p, pl.ds(sid * 10000, 10000)])


def _sc_layer(xvp, esrc, edst, et, z2, o2):
  mesh = plsc.VectorSubcoreMesh(core_axis_name="c", subcore_axis_name="s")
  f = pl.kernel(
      _sc_layer_kernel,
      out_type=(
          jax.ShapeDtypeStruct((16, NR, 8), jnp.float32),
          jax.ShapeDtypeStruct((NR, 8), jnp.float32),
      ),
      mesh=mesh,
      compiler_params=pltpu.CompilerParams(use_tc_tiling_on_sc=False),
      scratch_types=[
          pltpu.VMEM((CH,), jnp.int32),
          pltpu.VMEM((CH,), jnp.int32),
          pltpu.VMEM((NBP, 128), jnp.int32),
          pltpu.VMEM((NBP, 128), jnp.int32),
          pltpu.VMEM((4, 128, 8), jnp.float32),
          pltpu.VMEM_SHARED((NRP, 8), jnp.float32),
          pltpu.SemaphoreType.DMA,
          pltpu.SemaphoreType.DMA,
          pltpu.SemaphoreType.DMA,
          pltpu.SemaphoreType.DMA,
      ],
  )
  return f(xvp, esrc, edst, et, z2, o2)


def _wprep_kernel(comp_ref, basesf_ref, o_ref):
  o_ref[...] = jnp.dot(comp_ref[...], basesf_ref[...],
                       preferred_element_type=jnp.float32)


def _wprep(comp, basesf):
  return pl.pallas_call(
      _wprep_kernel,
      out_shape=jax.ShapeDtypeStruct((R, D * D), jnp.float32),
  )(comp, basesf)


BN = 400


def _tc_dense_kernel(relu, s_ref, c_ref, x_ref, w_ref, rep_ref, r_ref,
                     b_ref, o_ref):
  inv = 1.0 / jnp.maximum(c_ref[...], 1.0)
  inv128 = jnp.dot(inv, rep_ref[...],
                   preferred_element_type=jnp.float32)
  acc = jnp.dot(x_ref[...], r_ref[...], preferred_element_type=jnp.float32)
  for s in range(16):
    acc += jnp.dot(s_ref[s] * inv128, w_ref[s * D:(s + 1) * D, :],
                   preferred_element_type=jnp.float32)
  out = acc + b_ref[...]
  if relu:
    out = jnp.maximum(out, 0.0)
  o_ref[...] = out


def _tc_dense(sq, c16, x, wp, rep, root, bias2, relu):
  grid = (N // BN,)
  return pl.pallas_call(
      functools.partial(_tc_dense_kernel, relu),
      grid=grid,
      in_specs=[
          pl.BlockSpec((16, BN, D), lambda i: (0, i, 0)),
          pl.BlockSpec((BN, R), lambda i: (i, 0)),
          pl.BlockSpec((BN, D), lambda i: (i, 0)),
          pl.BlockSpec((16 * D, D), lambda i: (0, 0)),
          pl.BlockSpec((R, D), lambda i: (0, 0)),
          pl.BlockSpec((D, D), lambda i: (0, 0)),
          pl.BlockSpec((1, D), lambda i: (0, 0)),
      ],
      out_specs=pl.BlockSpec((BN, D), lambda i: (i, 0)),
      out_shape=jax.ShapeDtypeStruct((N, D), jnp.float32),
  )(sq, c16, x, wp, rep, root, bias2)


def _permute_w(wf):
  return (wf.reshape(R, 16, 8, D).transpose(1, 0, 2, 3)
          .reshape(16 * D, D))


def kernel(x, edge_index, edge_type, comp1, bases1, root1, bias1,
           comp2, bases2, root2, bias2):
  z2 = jnp.zeros((10000, 8), jnp.float32)
  o2 = jnp.ones((128, 8), jnp.float32)
  rep = (jnp.arange(D, dtype=jnp.int32)[None, :] // 8 ==
         jnp.arange(R, dtype=jnp.int32)[:, None]).astype(jnp.float32)

  wp1 = _permute_w(_wprep(comp1, bases1.reshape(B, D * D)).reshape(R * D, D))
  wp2 = _permute_w(_wprep(comp2, bases2.reshape(B, D * D)).reshape(R * D, D))
  b1 = bias1.reshape(1, D)
  b2 = bias2.reshape(1, D)

  npad = EP - E
  pad_i = jnp.arange(npad, dtype=jnp.int32)
  esrc = jnp.concatenate([edge_index[0], pad_i % N])
  edst = jnp.concatenate([edge_index[1], jnp.full((npad,), N, jnp.int32)])
  etp = jnp.concatenate([edge_type, pad_i % R])
  pad8 = jnp.zeros((8, 8), jnp.float32)

  xvp = jnp.concatenate([x.reshape(NR, 8), pad8])
  s1, c1 = _sc_layer(xvp, esrc, edst, etp, z2, o2)
  sq1 = s1.reshape(16, N, D)
  c16a = c1[:, 0].reshape(N, R)
  h = _tc_dense(sq1, c16a, x, wp1, rep, root1, b1, relu=True)

  hvp = jnp.concatenate([h.reshape(NR, 8), pad8])
  s2, c2 = _sc_layer(hvp, esrc, edst, etp, z2, o2)
  sq2 = s2.reshape(16, N, D)
  c16b = c2[:, 0].reshape(N, R)
  out = _tc_dense(sq2, c16b, h, wp2, rep, root2, b2, relu=False)
  return out
